# baseline (speedup 1.0000x reference)
import jax
import jax.numpy as jnp
import numpy as np
from jax.experimental import pallas as pl

_INV_EM1 = float(1.0 / (np.exp(1.0) - 1.0))
_MU = 3.5
_G = 25
_R = 1250
_L = 128


def _blend_block(p, v):
    lane = jax.lax.broadcasted_iota(jnp.int32, p.shape, 1)
    is_x = (lane & 1) == 0
    p_l = jnp.roll(p, -1, axis=1)
    p_r = jnp.roll(p, 1, axis=1)
    x = jnp.where(is_x, p, p_r)
    y = jnp.where(is_x, p_l, p)
    m = (x >= 0.0) & (x <= 0.25) & (y >= 0.0) & (y <= 1.0)
    xr = jnp.clip(x * 4.0, 0.0, 1.0)
    t = jnp.exp(jnp.log(xr) * _MU)
    g = (jnp.exp(t) - 1.0) * _INV_EM1
    g = jnp.minimum(g, 1.0)
    cs = jnp.where(is_x, 1.0, 0.0)
    v_out = jnp.where(m, v + g * (cs - v), v)
    return v_out, g


def _vel_kernel(pos_ref, vel_ref, velout_ref, gamma_ref):
    p = pos_ref[0]
    v = vel_ref[0]
    v_out, g = _blend_block(p, v)
    velout_ref[0] = v_out
    rows = jax.lax.broadcasted_iota(jnp.int32, (_L, _L // 2), 0)
    cols = jax.lax.broadcasted_iota(jnp.int32, (_L, _L // 2), 1)
    sel = (rows == 2 * cols).astype(jnp.float32)
    gamma_ref[0] = jax.lax.dot(g, sel, preferred_element_type=jnp.float32)


def kernel(fluidPosition, fluidVelocity, fluidArea):
    n = fluidPosition.shape[0]
    pos = fluidPosition.reshape(_G, _R, _L)
    vel = fluidVelocity.reshape(_G, _R, _L)
    vel_out, gamma = pl.pallas_call(
        _vel_kernel,
        grid=(_G,),
        in_specs=[
            pl.BlockSpec((1, _R, _L), lambda i: (i, 0, 0)),
            pl.BlockSpec((1, _R, _L), lambda i: (i, 0, 0)),
        ],
        out_specs=[
            pl.BlockSpec((1, _R, _L), lambda i: (i, 0, 0)),
            pl.BlockSpec((1, _R, _L // 2), lambda i: (i, 0, 0)),
        ],
        out_shape=[
            jax.ShapeDtypeStruct((_G, _R, _L), jnp.float32),
            jax.ShapeDtypeStruct((_G, _R, _L // 2), jnp.float32),
        ],
    )(pos, vel)
    return vel_out.reshape(n, 2), gamma.reshape(n)
